# baseline (speedup 1.0000x reference)
import functools

import jax
import jax.numpy as jnp
from jax import lax
from jax.experimental import pallas as pl
from jax.experimental.pallas import tpu as pltpu
from jax.experimental.pallas import tpu_sc as plsc

E = 8
HIDDEN = 1024
FFN = 3584
T = 2048
NA = 2 * T
B = 512
NB = (NA + E * B) // B
NPAD = NB * B
FT = 256
NFT = FFN // FT
CH = 512
NCH = NA // CH

NW = 32
APW = NA // NW
CHW = 64
TPW = T // NW


def _router_kernel(x_ref, gate_ref, dest_ref, v1_ref, v2_ref, bexp_ref,
                   nba_ref, xb_ref, excl_ref, m_ref):
    xb = x_ref[...].astype(jnp.bfloat16)
    xb_ref[...] = xb
    gb = gate_ref[...].astype(jnp.bfloat16)
    logits = jax.lax.dot_general(xb, gb, (((1,), (1,)), ((), ())),
                                 preferred_element_type=jnp.float32)
    m1 = jnp.max(logits, axis=1, keepdims=True)
    lane = jax.lax.broadcasted_iota(jnp.int32, logits.shape, 1)
    i1 = jnp.min(jnp.where(logits == m1, lane, E), axis=1, keepdims=True)
    masked = jnp.where(lane == i1, -jnp.inf, logits)
    m2 = jnp.max(masked, axis=1, keepdims=True)
    i2 = jnp.min(jnp.where(masked == m2, lane, E), axis=1, keepdims=True)
    r = jnp.exp(m2 - m1)
    v1_ref[...] = 1.0 / (1.0 + r)
    v2_ref[...] = 1.0 - v1_ref[...]

    ez = jnp.concatenate([i1, i2], axis=0)
    alane = jax.lax.broadcasted_iota(jnp.int32, (NA, E), 1)
    m_ref[...] = (alane == ez).astype(jnp.float32)

    ri = jax.lax.broadcasted_iota(jnp.int32, (CH, CH), 0)
    ci = jax.lax.broadcasted_iota(jnp.int32, (CH, CH), 1)
    trilb = (ri > ci).astype(jnp.bfloat16)

    def body(i, carry):
        mc = m_ref[pl.ds(i * CH, CH), :]
        part = jax.lax.dot_general(trilb, mc.astype(jnp.bfloat16),
                                   (((1,), (0,)), ((), ())),
                                   preferred_element_type=jnp.float32)
        excl_ref[pl.ds(i * CH, CH), :] = part + carry
        return carry + jnp.sum(mc, axis=0, keepdims=True)

    cnt = lax.fori_loop(0, NCH, body, jnp.zeros((1, E), jnp.float32))

    nblk = jnp.ceil(cnt / B)
    ei = jax.lax.broadcasted_iota(jnp.int32, (E, E), 0)
    ej = jax.lax.broadcasted_iota(jnp.int32, (E, E), 1)
    ltri = (ei < ej).astype(jnp.float32)
    bbase = jax.lax.dot_general(nblk, ltri, (((1,), (0,)), ((), ())),
                                preferred_element_type=jnp.float32)
    base = bbase * B

    dest = jnp.sum(m_ref[...] * (base + excl_ref[...]), axis=1, keepdims=True)
    dest_ref[...] = dest.astype(jnp.int32)

    nba = bbase[0, E - 1] + nblk[0, E - 1]
    bi = jax.lax.broadcasted_iota(jnp.int32, (NB, E), 0).astype(jnp.float32)
    el = jax.lax.broadcasted_iota(jnp.int32, (NB, E), 1)
    own = jnp.logical_and(bi >= bbase, bi < bbase + nblk)
    bexp = jnp.sum(jnp.where(own, el, 0), axis=1, keepdims=True)
    has = jnp.sum(own.astype(jnp.int32), axis=1, keepdims=True)
    bexp_ref[...] = jnp.where(has > 0, bexp, E - 1)
    nba_ref[...] = jnp.full((1, 1), nba, jnp.float32).astype(jnp.int32)


def _run_router(x, gate_w):
    return pl.pallas_call(
        _router_kernel,
        in_specs=[pl.BlockSpec((T, HIDDEN), lambda: (0, 0)),
                  pl.BlockSpec((E, HIDDEN), lambda: (0, 0))],
        out_specs=[pl.BlockSpec((NA, 1), lambda: (0, 0)),
                   pl.BlockSpec((T, 1), lambda: (0, 0)),
                   pl.BlockSpec((T, 1), lambda: (0, 0)),
                   pl.BlockSpec((NB, 1), lambda: (0, 0)),
                   pl.BlockSpec((1, 1), lambda: (0, 0)),
                   pl.BlockSpec((T, HIDDEN), lambda: (0, 0))],
        out_shape=[jax.ShapeDtypeStruct((NA, 1), jnp.int32),
                   jax.ShapeDtypeStruct((T, 1), jnp.float32),
                   jax.ShapeDtypeStruct((T, 1), jnp.float32),
                   jax.ShapeDtypeStruct((NB, 1), jnp.int32),
                   jax.ShapeDtypeStruct((1, 1), jnp.int32),
                   jax.ShapeDtypeStruct((T, HIDDEN), jnp.bfloat16)],
        scratch_shapes=[pltpu.VMEM((NA, E), jnp.float32),
                        pltpu.VMEM((NA, E), jnp.float32)],
    )(x, gate_w)


def _dispatch_sc(x, dest):
    mesh = plsc.VectorSubcoreMesh(core_axis_name="c", subcore_axis_name="s")

    @functools.partial(
        pl.kernel, mesh=mesh,
        out_type=jax.ShapeDtypeStruct((NPAD, HIDDEN), jnp.float32),
        scratch_types=[pltpu.VMEM((1, CHW), jnp.int32),
                       pltpu.VMEM((CHW, HIDDEN), jnp.float32)],
    )
    def k(x_hbm, dest_hbm, xs_hbm, idx_v, rows_v):
        wid = lax.axis_index("s") * 2 + lax.axis_index("c")
        for c in range(APW // CHW):
            abase = wid * APW + c * CHW
            tbase = lax.rem(abase, T)
            pltpu.sync_copy(dest_hbm.at[pl.ds(abase, CHW)], idx_v.at[0])
            pltpu.sync_copy(x_hbm.at[pl.ds(tbase, CHW)], rows_v)
            pltpu.sync_copy(rows_v, xs_hbm.at[idx_v.at[0]])

    return k(x, dest)


def _mm_kernel(bexp_ref, nba_ref, xs_ref, w1_ref, w3_ref, w2_ref, out_ref,
               acc_ref, xsb_ref):
    ft = pl.program_id(0)
    b = pl.program_id(1)
    active = b < nba_ref[0]

    @pl.when(jnp.logical_and(active, ft == 0))
    def _():
        xsb_ref[pl.ds(b * B, B), :] = xs_ref[...].astype(jnp.bfloat16)

    @pl.when(active)
    def _():
        xb = xsb_ref[pl.ds(b * B, B), :]
        w1b = w1_ref[0].astype(jnp.bfloat16)
        w3b = w3_ref[0].astype(jnp.bfloat16)
        h = jax.lax.dot_general(xb, w1b, (((1,), (1,)), ((), ())),
                                preferred_element_type=jnp.float32)
        u = jax.lax.dot_general(xb, w3b, (((1,), (1,)), ((), ())),
                                preferred_element_type=jnp.float32)
        a = (h * jax.nn.sigmoid(h) * u).astype(jnp.bfloat16)
        w2b = w2_ref[0].astype(jnp.bfloat16)
        part = jax.lax.dot_general(a, w2b, (((1,), (1,)), ((), ())),
                                   preferred_element_type=jnp.float32)

        @pl.when(ft == 0)
        def _():
            acc_ref[pl.ds(b * B, B), :] = part.astype(jnp.bfloat16)

        @pl.when(jnp.logical_and(ft != 0, ft != NFT - 1))
        def _():
            sl = acc_ref[pl.ds(b * B, B), :].astype(jnp.float32)
            acc_ref[pl.ds(b * B, B), :] = (sl + part).astype(jnp.bfloat16)

        @pl.when(ft == NFT - 1)
        def _():
            out_ref[...] = acc_ref[pl.ds(b * B, B), :].astype(jnp.float32) + part


def _run_mm(bexp, nba, xs, w1, w3, w2):
    grid_spec = pltpu.PrefetchScalarGridSpec(
        num_scalar_prefetch=2,
        grid=(NFT, NB),
        in_specs=[
            pl.BlockSpec((B, HIDDEN),
                         lambda ft, b, be, na: (
                             jnp.where(ft == 0, jnp.minimum(b, na[0] - 1),
                                       na[0] - 1), 0)),
            pl.BlockSpec((1, FT, HIDDEN), lambda ft, b, be, na: (be[b], ft, 0)),
            pl.BlockSpec((1, FT, HIDDEN), lambda ft, b, be, na: (be[b], ft, 0)),
            pl.BlockSpec((1, HIDDEN, FT), lambda ft, b, be, na: (be[b], 0, ft)),
        ],
        out_specs=pl.BlockSpec(
            (B, HIDDEN),
            lambda ft, b, be, na: (
                jnp.where(jnp.logical_and(ft == NFT - 1, b < na[0]), b, NB), 0)),
        scratch_shapes=[pltpu.VMEM((NPAD, HIDDEN), jnp.bfloat16),
                        pltpu.VMEM((NPAD, HIDDEN), jnp.bfloat16)],
    )
    return pl.pallas_call(
        _mm_kernel,
        grid_spec=grid_spec,
        out_shape=jax.ShapeDtypeStruct(((NB + 1) * B, HIDDEN), jnp.float32),
        compiler_params=pltpu.CompilerParams(
            dimension_semantics=("arbitrary", "arbitrary"),
        ),
    )(bexp, nba, xs, w1, w3, w2)


def _gather_sc(os_big, dest):
    mesh = plsc.VectorSubcoreMesh(core_axis_name="c", subcore_axis_name="s")
    otype = jax.ShapeDtypeStruct((T, HIDDEN), jnp.float32)

    @functools.partial(
        pl.kernel, mesh=mesh, out_type=[otype, otype],
        scratch_types=[pltpu.VMEM((CHW,), jnp.int32),
                       pltpu.VMEM((CHW, HIDDEN), jnp.float32)],
    )
    def k(os_hbm, dest_hbm, o0_hbm, o1_hbm, idx_v, rows_v):
        wid = lax.axis_index("s") * 2 + lax.axis_index("c")
        tbase = wid * TPW
        for kk, o_hbm in enumerate((o0_hbm, o1_hbm)):
            pltpu.sync_copy(dest_hbm.at[pl.ds(kk * T + tbase, CHW)], idx_v)
            pltpu.sync_copy(os_hbm.at[idx_v], rows_v)
            pltpu.sync_copy(rows_v, o_hbm.at[pl.ds(tbase, CHW)])

    return k(os_big, dest)


def _combine_kernel(o0_ref, o1_ref, v1_ref, v2_ref, out_ref):
    out_ref[...] = v1_ref[...] * o0_ref[...] + v2_ref[...] * o1_ref[...]


def _run_combine(o0, o1, v1, v2):
    nblk = 4
    rb = T // nblk
    return pl.pallas_call(
        _combine_kernel,
        grid=(nblk,),
        in_specs=[pl.BlockSpec((rb, HIDDEN), lambda i: (i, 0)),
                  pl.BlockSpec((rb, HIDDEN), lambda i: (i, 0)),
                  pl.BlockSpec((rb, 1), lambda i: (i, 0)),
                  pl.BlockSpec((rb, 1), lambda i: (i, 0))],
        out_specs=pl.BlockSpec((rb, HIDDEN), lambda i: (i, 0)),
        out_shape=jax.ShapeDtypeStruct((T, HIDDEN), jnp.float32),
    )(o0, o1, v1, v2)


@jax.jit
def kernel(hidden_states, gate_w, w1, w2, w3):
    x = hidden_states.reshape(T, HIDDEN)
    dest, v1, v2, bexp, nba, _xb16 = _run_router(x, gate_w)
    dest_flat = dest.reshape(NA)
    xs = _dispatch_sc(x, dest_flat)
    os_big = _run_mm(bexp.reshape(NB), nba.reshape(1), xs, w1, w3, w2)
    o0, o1 = _gather_sc(os_big, dest_flat)
    out = _run_combine(o0, o1, v1, v2)
    return out.reshape(hidden_states.shape)


# speedup vs baseline: 1.1952x; 1.1952x over previous
import functools

import jax
import jax.numpy as jnp
from jax import lax
from jax.experimental import pallas as pl
from jax.experimental.pallas import tpu as pltpu
from jax.experimental.pallas import tpu_sc as plsc

E = 8
HIDDEN = 1024
FFN = 3584
T = 2048
NA = 2 * T
B = 512
NB = (NA + E * B) // B
NPAD = NB * B
FT = 512
NFT = FFN // FT
CH = 512
NCH = NA // CH

NW = 32
APW = NA // NW
CHW = 64
TPW = T // NW


def _router_kernel(x_ref, gate_ref, dest_ref, v1_ref, v2_ref, bexp_ref,
                   nba_ref, xb_ref, excl_ref, m_ref):
    xb = x_ref[...].astype(jnp.bfloat16)
    xb_ref[...] = xb
    gb = gate_ref[...].astype(jnp.bfloat16)
    logits = jax.lax.dot_general(xb, gb, (((1,), (1,)), ((), ())),
                                 preferred_element_type=jnp.float32)
    m1 = jnp.max(logits, axis=1, keepdims=True)
    lane = jax.lax.broadcasted_iota(jnp.int32, logits.shape, 1)
    i1 = jnp.min(jnp.where(logits == m1, lane, E), axis=1, keepdims=True)
    masked = jnp.where(lane == i1, -jnp.inf, logits)
    m2 = jnp.max(masked, axis=1, keepdims=True)
    i2 = jnp.min(jnp.where(masked == m2, lane, E), axis=1, keepdims=True)
    r = jnp.exp(m2 - m1)
    v1_ref[...] = 1.0 / (1.0 + r)
    v2_ref[...] = 1.0 - v1_ref[...]

    ez = jnp.concatenate([i1, i2], axis=0)
    alane = jax.lax.broadcasted_iota(jnp.int32, (NA, E), 1)
    m_ref[...] = (alane == ez).astype(jnp.float32)

    ri = jax.lax.broadcasted_iota(jnp.int32, (CH, CH), 0)
    ci = jax.lax.broadcasted_iota(jnp.int32, (CH, CH), 1)
    trilb = (ri > ci).astype(jnp.bfloat16)

    def body(i, carry):
        mc = m_ref[pl.ds(i * CH, CH), :]
        part = jax.lax.dot_general(trilb, mc.astype(jnp.bfloat16),
                                   (((1,), (0,)), ((), ())),
                                   preferred_element_type=jnp.float32)
        excl_ref[pl.ds(i * CH, CH), :] = part + carry
        return carry + jnp.sum(mc, axis=0, keepdims=True)

    cnt = lax.fori_loop(0, NCH, body, jnp.zeros((1, E), jnp.float32))

    nblk = jnp.ceil(cnt / B)
    ei = jax.lax.broadcasted_iota(jnp.int32, (E, E), 0)
    ej = jax.lax.broadcasted_iota(jnp.int32, (E, E), 1)
    ltri = (ei < ej).astype(jnp.float32)
    bbase = jax.lax.dot_general(nblk, ltri, (((1,), (0,)), ((), ())),
                                preferred_element_type=jnp.float32)
    base = bbase * B

    dest = jnp.sum(m_ref[...] * (base + excl_ref[...]), axis=1, keepdims=True)
    dest_ref[...] = dest.astype(jnp.int32)

    nba = bbase[0, E - 1] + nblk[0, E - 1]
    bi = jax.lax.broadcasted_iota(jnp.int32, (NB, E), 0).astype(jnp.float32)
    el = jax.lax.broadcasted_iota(jnp.int32, (NB, E), 1)
    own = jnp.logical_and(bi >= bbase, bi < bbase + nblk)
    bexp = jnp.sum(jnp.where(own, el, 0), axis=1, keepdims=True)
    has = jnp.sum(own.astype(jnp.int32), axis=1, keepdims=True)
    bexp_ref[...] = jnp.where(has > 0, bexp, E - 1)
    nba_ref[...] = jnp.full((1, 1), nba, jnp.float32).astype(jnp.int32)


def _run_router(x, gate_w):
    return pl.pallas_call(
        _router_kernel,
        in_specs=[pl.BlockSpec((T, HIDDEN), lambda: (0, 0)),
                  pl.BlockSpec((E, HIDDEN), lambda: (0, 0))],
        out_specs=[pl.BlockSpec((NA, 1), lambda: (0, 0)),
                   pl.BlockSpec((T, 1), lambda: (0, 0)),
                   pl.BlockSpec((T, 1), lambda: (0, 0)),
                   pl.BlockSpec((NB, 1), lambda: (0, 0)),
                   pl.BlockSpec((1, 1), lambda: (0, 0)),
                   pl.BlockSpec((T, HIDDEN), lambda: (0, 0))],
        out_shape=[jax.ShapeDtypeStruct((NA, 1), jnp.int32),
                   jax.ShapeDtypeStruct((T, 1), jnp.float32),
                   jax.ShapeDtypeStruct((T, 1), jnp.float32),
                   jax.ShapeDtypeStruct((NB, 1), jnp.int32),
                   jax.ShapeDtypeStruct((1, 1), jnp.int32),
                   jax.ShapeDtypeStruct((T, HIDDEN), jnp.bfloat16)],
        scratch_shapes=[pltpu.VMEM((NA, E), jnp.float32),
                        pltpu.VMEM((NA, E), jnp.float32)],
    )(x, gate_w)


def _dispatch_sc(x, dest):
    mesh = plsc.VectorSubcoreMesh(core_axis_name="c", subcore_axis_name="s")

    @functools.partial(
        pl.kernel, mesh=mesh,
        out_type=jax.ShapeDtypeStruct((NPAD, HIDDEN), jnp.float32),
        scratch_types=[pltpu.VMEM((1, CHW), jnp.int32),
                       pltpu.VMEM((CHW, HIDDEN), jnp.float32)],
    )
    def k(x_hbm, dest_hbm, xs_hbm, idx_v, rows_v):
        wid = lax.axis_index("s") * 2 + lax.axis_index("c")
        for c in range(APW // CHW):
            abase = wid * APW + c * CHW
            tbase = lax.rem(abase, T)
            pltpu.sync_copy(dest_hbm.at[pl.ds(abase, CHW)], idx_v.at[0])
            pltpu.sync_copy(x_hbm.at[pl.ds(tbase, CHW)], rows_v)
            pltpu.sync_copy(rows_v, xs_hbm.at[idx_v.at[0]])

    return k(x, dest)


def _mm_kernel(bexp_ref, nba_ref, xs_ref, w1_ref, w3_ref, w2_ref, out_ref,
               acc_ref, xsb_ref):
    ft = pl.program_id(0)
    b = pl.program_id(1)
    active = b < nba_ref[0]

    @pl.when(jnp.logical_and(active, ft == 0))
    def _():
        xsb_ref[pl.ds(b * B, B), :] = xs_ref[...].astype(jnp.bfloat16)

    @pl.when(active)
    def _():
        xb = xsb_ref[pl.ds(b * B, B), :]
        w1b = w1_ref[0].astype(jnp.bfloat16)
        w3b = w3_ref[0].astype(jnp.bfloat16)
        h = jax.lax.dot_general(xb, w1b, (((1,), (1,)), ((), ())),
                                preferred_element_type=jnp.float32)
        u = jax.lax.dot_general(xb, w3b, (((1,), (1,)), ((), ())),
                                preferred_element_type=jnp.float32)
        a = (h * jax.nn.sigmoid(h) * u).astype(jnp.bfloat16)
        w2b = w2_ref[0].astype(jnp.bfloat16)
        part = jax.lax.dot_general(a, w2b, (((1,), (1,)), ((), ())),
                                   preferred_element_type=jnp.float32)

        @pl.when(ft == 0)
        def _():
            acc_ref[pl.ds(b * B, B), :] = part.astype(jnp.bfloat16)

        @pl.when(jnp.logical_and(ft != 0, ft != NFT - 1))
        def _():
            sl = acc_ref[pl.ds(b * B, B), :].astype(jnp.float32)
            acc_ref[pl.ds(b * B, B), :] = (sl + part).astype(jnp.bfloat16)

        @pl.when(ft == NFT - 1)
        def _():
            out_ref[...] = acc_ref[pl.ds(b * B, B), :].astype(jnp.float32) + part


def _run_mm(bexp, nba, xs, w1, w3, w2):
    grid_spec = pltpu.PrefetchScalarGridSpec(
        num_scalar_prefetch=2,
        grid=(NFT, NB),
        in_specs=[
            pl.BlockSpec((B, HIDDEN),
                         lambda ft, b, be, na: (
                             jnp.where(ft == 0, jnp.minimum(b, na[0] - 1),
                                       na[0] - 1), 0)),
            pl.BlockSpec((1, FT, HIDDEN), lambda ft, b, be, na: (be[b], ft, 0)),
            pl.BlockSpec((1, FT, HIDDEN), lambda ft, b, be, na: (be[b], ft, 0)),
            pl.BlockSpec((1, HIDDEN, FT), lambda ft, b, be, na: (be[b], 0, ft)),
        ],
        out_specs=pl.BlockSpec(
            (B, HIDDEN),
            lambda ft, b, be, na: (
                jnp.where(jnp.logical_and(ft == NFT - 1, b < na[0]), b, NB), 0)),
        scratch_shapes=[pltpu.VMEM((NPAD, HIDDEN), jnp.bfloat16),
                        pltpu.VMEM((NPAD, HIDDEN), jnp.bfloat16)],
    )
    return pl.pallas_call(
        _mm_kernel,
        grid_spec=grid_spec,
        out_shape=jax.ShapeDtypeStruct(((NB + 1) * B, HIDDEN), jnp.float32),
        compiler_params=pltpu.CompilerParams(
            dimension_semantics=("arbitrary", "arbitrary"),
        ),
    )(bexp, nba, xs, w1, w3, w2)


def _gather_sc(os_big, dest):
    mesh = plsc.VectorSubcoreMesh(core_axis_name="c", subcore_axis_name="s")
    otype = jax.ShapeDtypeStruct((T, HIDDEN), jnp.float32)

    @functools.partial(
        pl.kernel, mesh=mesh, out_type=[otype, otype],
        scratch_types=[pltpu.VMEM((CHW,), jnp.int32),
                       pltpu.VMEM((CHW, HIDDEN), jnp.float32)],
    )
    def k(os_hbm, dest_hbm, o0_hbm, o1_hbm, idx_v, rows_v):
        wid = lax.axis_index("s") * 2 + lax.axis_index("c")
        tbase = wid * TPW
        for kk, o_hbm in enumerate((o0_hbm, o1_hbm)):
            pltpu.sync_copy(dest_hbm.at[pl.ds(kk * T + tbase, CHW)], idx_v)
            pltpu.sync_copy(os_hbm.at[idx_v], rows_v)
            pltpu.sync_copy(rows_v, o_hbm.at[pl.ds(tbase, CHW)])

    return k(os_big, dest)


def _combine_kernel(o0_ref, o1_ref, v1_ref, v2_ref, out_ref):
    out_ref[...] = v1_ref[...] * o0_ref[...] + v2_ref[...] * o1_ref[...]


def _run_combine(o0, o1, v1, v2):
    nblk = 4
    rb = T // nblk
    return pl.pallas_call(
        _combine_kernel,
        grid=(nblk,),
        in_specs=[pl.BlockSpec((rb, HIDDEN), lambda i: (i, 0)),
                  pl.BlockSpec((rb, HIDDEN), lambda i: (i, 0)),
                  pl.BlockSpec((rb, 1), lambda i: (i, 0)),
                  pl.BlockSpec((rb, 1), lambda i: (i, 0))],
        out_specs=pl.BlockSpec((rb, HIDDEN), lambda i: (i, 0)),
        out_shape=jax.ShapeDtypeStruct((T, HIDDEN), jnp.float32),
    )(o0, o1, v1, v2)


@jax.jit
def kernel(hidden_states, gate_w, w1, w2, w3):
    x = hidden_states.reshape(T, HIDDEN)
    dest, v1, v2, bexp, nba, _xb16 = _run_router(x, gate_w)
    dest_flat = dest.reshape(NA)
    xs = _dispatch_sc(x, dest_flat)
    os_big = _run_mm(bexp.reshape(NB), nba.reshape(1), xs, w1, w3, w2)
    o0, o1 = _gather_sc(os_big, dest_flat)
    out = _run_combine(o0, o1, v1, v2)
    return out.reshape(hidden_states.shape)
